# Initial kernel scaffold; baseline (speedup 1.0000x reference)
import functools

import jax
import jax.numpy as jnp
from jax import lax
from jax.experimental import pallas as pl
from jax.experimental.pallas import tpu as pltpu
from jax.experimental.pallas import tpu_sc as plsc


def _topk_body(K, N, cq_ref, ck_ref, idx_ref):
    b = pl.program_id(0)
    cq = cq_ref[0]
    ck = ck_ref[0]
    sqq = jnp.sum(cq * cq, axis=1, keepdims=True)
    sqk = jnp.sum(ck * ck, axis=0, keepdims=True)
    d2 = sqq + sqk - 2.0 * jnp.dot(cq, ck, preferred_element_type=jnp.float32)
    d2 = jnp.maximum(d2, 0.0)
    Q = d2.shape[0]
    col = lax.broadcasted_iota(jnp.int32, (Q, N), 1)
    big = jnp.float32(3.4e38)
    js = []
    for _ in range(K):
        m = jnp.min(d2, axis=1, keepdims=True)
        cand = jnp.where(d2 == m, col, N)
        j = jnp.min(cand, axis=1, keepdims=True)
        js.append(j)
        d2 = jnp.where(col == j, big, d2)
    idx_ref[0] = jnp.concatenate(js, axis=1) + b * N


def _topk(coords_pad, coords_t, K, Q):
    B, N, _ = coords_pad.shape
    grid = (B, N // Q)
    return pl.pallas_call(
        functools.partial(_topk_body, K, N),
        grid=grid,
        in_specs=[
            pl.BlockSpec((1, Q, 8), lambda b, i: (b, i, 0)),
            pl.BlockSpec((1, 8, N), lambda b, i: (b, 0, 0)),
        ],
        out_specs=pl.BlockSpec((1, Q, K), lambda b, i: (b, i, 0)),
        out_shape=jax.ShapeDtypeStruct((B, N, K), jnp.int32),
    )(coords_pad, coords_t)


def _sc_gather(table, gidx):
    info = plsc.get_sparse_core_info()
    NC, NS = info.num_cores, info.num_subcores
    NW = NC * NS
    _, C, _ = gidx.shape
    rows_per_w = C * 128
    R = NW * rows_per_w
    mesh = plsc.VectorSubcoreMesh(core_axis_name="c", subcore_axis_name="s")

    @functools.partial(
        pl.kernel,
        mesh=mesh,
        out_type=jax.ShapeDtypeStruct((R, 16), jnp.float32),
        scratch_types=[
            pltpu.VMEM((C, 128), jnp.int32),
            pltpu.VMEM((rows_per_w, 16), jnp.float32),
            pltpu.SemaphoreType.DMA,
        ],
    )
    def k2(table_hbm, idx_hbm, out_hbm, idx_v, rows_v, sem):
        wid = lax.axis_index("s") * NC + lax.axis_index("c")
        pltpu.sync_copy(idx_hbm.at[wid], idx_v)

        def chunk(c, carry):
            pltpu.async_copy(
                table_hbm.at[idx_v.at[c]],
                rows_v.at[pl.ds(c * 128, 128)],
                sem,
            ).wait()
            return carry

        lax.fori_loop(0, C, chunk, 0)
        pltpu.sync_copy(rows_v, out_hbm.at[pl.ds(wid * rows_per_w, rows_per_w)])

    return k2(table, gidx)


def _ln(h, g, b, eps=1e-5):
    mu = jnp.mean(h, axis=-1, keepdims=True)
    var = jnp.mean((h - mu) ** 2, axis=-1, keepdims=True)
    return (h - mu) / jnp.sqrt(var + eps) * g + b


def _safe_atan2(y, x):
    both_zero = (jnp.abs(x) < 1e-9) & (jnp.abs(y) < 1e-9)
    x_safe = jnp.where(both_zero, 1.0, x)
    y_safe = jnp.where(both_zero, 0.0, y)
    return jnp.arctan2(y_safe, x_safe)


def _mlp_body(K, nbr_ref, ctr_ref,
              w1_ref, b1_ref, g1_ref, be1_ref,
              w2_ref, b2_ref, g2_ref, be2_ref,
              w3_ref, b3_ref, g3_ref, be3_ref,
              wa1_ref, ba1_ref, ga1_ref, bea1_ref,
              wa2_ref, ba2_ref, out_ref):
    nb = nbr_ref[...]
    ctr = ctr_ref[...]
    RT = nb.shape[0]
    PT = ctr.shape[0]

    re = lax.broadcasted_iota(jnp.int32, (RT, PT), 0) // K
    ce = lax.broadcasted_iota(jnp.int32, (RT, PT), 1)
    E = (re == ce).astype(jnp.float32)
    ctr_rows = jnp.dot(E, ctr, preferred_element_type=jnp.float32)

    rel = nb - ctr_rows
    d2 = jnp.sum(rel * rel, axis=1, keepdims=True)
    dist = jnp.sqrt(d2 + 1e-12)
    reln = rel / (dist + 1e-6)

    lane = lax.broadcasted_iota(jnp.int32, (RT, 16), 1)

    def col(a, c):
        return jnp.sum(jnp.where(lane == c, a, 0.0), axis=1, keepdims=True)

    rx, ry, rz = col(rel, 0), col(rel, 1), col(rel, 2)
    nx, ny, nz = col(reln, 0), col(reln, 1), col(reln, 2)
    axy = _safe_atan2(ny, nx)
    axz = _safe_atan2(nz, nx)
    ayz = _safe_atan2(nz, ny)

    feat = (jnp.where(lane == 0, dist, 0.0)
            + jnp.where(lane == 1, rx, 0.0)
            + jnp.where(lane == 2, ry, 0.0)
            + jnp.where(lane == 3, rz, 0.0)
            + jnp.where(lane == 4, axy, 0.0)
            + jnp.where(lane == 5, axz, 0.0)
            + jnp.where(lane == 6, ayz, 0.0)
            + jnp.where(lane == 7, nx, 0.0)
            + jnp.where(lane == 8, ny, 0.0)
            + jnp.where(lane == 9, nz, 0.0))

    h = jnp.dot(feat, w1_ref[...], preferred_element_type=jnp.float32)
    h = jnp.maximum(_ln(h + b1_ref[...], g1_ref[...], be1_ref[...]), 0.0)
    h = jnp.dot(h, w2_ref[...], preferred_element_type=jnp.float32)
    h = jnp.maximum(_ln(h + b2_ref[...], g2_ref[...], be2_ref[...]), 0.0)
    h = jnp.dot(h, w3_ref[...], preferred_element_type=jnp.float32)
    h = _ln(h + b3_ref[...], g3_ref[...], be3_ref[...])

    P = jnp.transpose(E) * (1.0 / K)
    agg = jnp.dot(P, h, preferred_element_type=jnp.float32)

    a = jnp.dot(agg, wa1_ref[...], preferred_element_type=jnp.float32)
    a = jnp.maximum(_ln(a + ba1_ref[...], ga1_ref[...], bea1_ref[...]), 0.0)
    out_ref[...] = (jnp.dot(a, wa2_ref[...], preferred_element_type=jnp.float32)
                    + ba2_ref[...])


def _mlp(nbr, table, K, PT, weights):
    (W1p, b1, g1, be1, W2, b2, g2, be2, W3, b3, g3, be3,
     Wa1, ba1, ga1, bea1, Wa2, ba2) = weights
    BN = table.shape[0]
    RT = PT * K
    grid = (BN // PT,)

    def full(a):
        return pl.BlockSpec(a.shape, lambda i: (0,) * a.ndim)

    return pl.pallas_call(
        functools.partial(_mlp_body, K),
        grid=grid,
        in_specs=[
            pl.BlockSpec((RT, 16), lambda i: (i, 0)),
            pl.BlockSpec((PT, 16), lambda i: (i, 0)),
            full(W1p), full(b1), full(g1), full(be1),
            full(W2), full(b2), full(g2), full(be2),
            full(W3), full(b3), full(g3), full(be3),
            full(Wa1), full(ba1), full(ga1), full(bea1),
            full(Wa2), full(ba2),
        ],
        out_specs=pl.BlockSpec((PT, 128), lambda i: (i, 0)),
        out_shape=jax.ShapeDtypeStruct((BN, 128), jnp.float32),
    )(nbr, table, W1p, b1, g1, be1, W2, b2, g2, be2, W3, b3, g3, be3,
      Wa1, ba1, ga1, bea1, Wa2, ba2)


def kernel(coordinates, W1, b1, g1, be1, W2, b2, g2, be2, W3, b3, g3, be3,
           Wa1, ba1, ga1, bea1, Wa2, ba2):
    K = 16
    B, N, _ = coordinates.shape
    D = W3.shape[1]
    Q = 256 if N % 256 == 0 else N

    cpad = jnp.pad(coordinates, ((0, 0), (0, 0), (0, 5)))
    ct = jnp.transpose(cpad, (0, 2, 1))
    idx = _topk(cpad, ct, K, Q)

    table = jnp.pad(coordinates.reshape(B * N, 3), ((0, 0), (0, 13)))
    NW = 32
    gidx = idx.reshape(NW, (B * N * K) // (NW * 128), 128)
    nbr = _sc_gather(table, gidx)

    W1p = jnp.pad(W1, ((0, 6), (0, 0)))
    weights = (W1p, b1.reshape(1, -1), g1.reshape(1, -1), be1.reshape(1, -1),
               W2, b2.reshape(1, -1), g2.reshape(1, -1), be2.reshape(1, -1),
               W3, b3.reshape(1, -1), g3.reshape(1, -1), be3.reshape(1, -1),
               Wa1, ba1.reshape(1, -1), ga1.reshape(1, -1), bea1.reshape(1, -1),
               Wa2, ba2.reshape(1, -1))
    out = _mlp(nbr, table, K, 64, weights)
    return out.reshape(B, N, D)


# speedup vs baseline: 6.2424x; 6.2424x over previous
import functools

import jax
import jax.numpy as jnp
from jax import lax
from jax.experimental import pallas as pl
from jax.experimental.pallas import tpu as pltpu
from jax.experimental.pallas import tpu_sc as plsc


def _topk_body(K, N, cq_ref, ck_ref, idx_ref):
    b = pl.program_id(0)
    cq = cq_ref[0]
    ck = ck_ref[0]
    sqq = jnp.sum(cq * cq, axis=1, keepdims=True)
    sqk = jnp.sum(ck * ck, axis=0, keepdims=True)
    d2 = sqq + sqk - 2.0 * jnp.dot(cq, ck, preferred_element_type=jnp.float32)
    d2 = jnp.maximum(d2, 0.0)
    Q = d2.shape[0]
    col = lax.broadcasted_iota(jnp.int32, (Q, N), 1)
    big = jnp.float32(3.4e38)
    js = []
    for _ in range(K):
        m = jnp.min(d2, axis=1, keepdims=True)
        cand = jnp.where(d2 == m, col, N)
        j = jnp.min(cand, axis=1, keepdims=True)
        js.append(j)
        d2 = jnp.where(col == j, big, d2)
    idx_ref[0] = jnp.concatenate(js, axis=1) + b * N


def _topk(coords_pad, coords_t, K, Q):
    B, N, _ = coords_pad.shape
    grid = (B, N // Q)
    return pl.pallas_call(
        functools.partial(_topk_body, K, N),
        grid=grid,
        in_specs=[
            pl.BlockSpec((1, Q, 8), lambda b, i: (b, i, 0)),
            pl.BlockSpec((1, 8, N), lambda b, i: (b, 0, 0)),
        ],
        out_specs=pl.BlockSpec((1, Q, K), lambda b, i: (b, i, 0)),
        out_shape=jax.ShapeDtypeStruct((B, N, K), jnp.int32),
    )(coords_pad, coords_t)


def _sc_gather(table, gidx):
    info = plsc.get_sparse_core_info()
    NC, NS = info.num_cores, info.num_subcores
    NW = NC * NS
    _, C, _ = gidx.shape
    rows_per_w = C * 128
    R = NW * rows_per_w
    mesh = plsc.VectorSubcoreMesh(core_axis_name="c", subcore_axis_name="s")

    @functools.partial(
        pl.kernel,
        mesh=mesh,
        compiler_params=pltpu.CompilerParams(use_tc_tiling_on_sc=False),
        out_type=jax.ShapeDtypeStruct((R, 16), jnp.float32),
        scratch_types=[
            pltpu.VMEM((C, 128), jnp.int32),
            pltpu.VMEM((rows_per_w, 16), jnp.float32),
            pltpu.SemaphoreType.DMA,
        ],
    )
    def k2(table_hbm, idx_hbm, out_hbm, idx_v, rows_v, sem):
        wid = lax.axis_index("s") * NC + lax.axis_index("c")
        pltpu.sync_copy(idx_hbm.at[wid], idx_v)

        def chunk(c, carry):
            pltpu.async_copy(
                table_hbm.at[idx_v.at[c]],
                rows_v.at[pl.ds(c * 128, 128)],
                sem,
            ).wait()
            return carry

        lax.fori_loop(0, C, chunk, 0)
        pltpu.sync_copy(rows_v, out_hbm.at[pl.ds(wid * rows_per_w, rows_per_w)])

    return k2(table, gidx)


def _ln(h, g, b, eps=1e-5):
    mu = jnp.mean(h, axis=-1, keepdims=True)
    var = jnp.mean((h - mu) ** 2, axis=-1, keepdims=True)
    return (h - mu) / jnp.sqrt(var + eps) * g + b


def _safe_atan2(y, x):
    both_zero = (jnp.abs(x) < 1e-9) & (jnp.abs(y) < 1e-9)
    x_safe = jnp.where(both_zero, 1.0, x)
    y_safe = jnp.where(both_zero, 0.0, y)
    return jnp.arctan2(y_safe, x_safe)


def _mlp_body(K, nbr_ref, ctr_ref,
              w1_ref, b1_ref, g1_ref, be1_ref,
              w2_ref, b2_ref, g2_ref, be2_ref,
              w3_ref, b3_ref, g3_ref, be3_ref,
              wa1_ref, ba1_ref, ga1_ref, bea1_ref,
              wa2_ref, ba2_ref, out_ref):
    nb = nbr_ref[...]
    ctr = ctr_ref[...]
    RT = nb.shape[0]
    PT = ctr.shape[0]

    re = lax.broadcasted_iota(jnp.int32, (RT, PT), 0) // K
    ce = lax.broadcasted_iota(jnp.int32, (RT, PT), 1)
    E = (re == ce).astype(jnp.float32)
    ctr_rows = jnp.dot(E, ctr, preferred_element_type=jnp.float32,
                       precision=jax.lax.Precision.HIGHEST)

    rel = nb - ctr_rows
    d2 = jnp.sum(rel * rel, axis=1, keepdims=True)
    dist = jnp.sqrt(d2 + 1e-12)
    reln = rel / (dist + 1e-6)

    lane = lax.broadcasted_iota(jnp.int32, (RT, 16), 1)

    def col(a, c):
        return jnp.sum(jnp.where(lane == c, a, 0.0), axis=1, keepdims=True)

    rx, ry, rz = col(rel, 0), col(rel, 1), col(rel, 2)
    nx, ny, nz = col(reln, 0), col(reln, 1), col(reln, 2)
    axy = _safe_atan2(ny, nx)
    axz = _safe_atan2(nz, nx)
    ayz = _safe_atan2(nz, ny)

    feat = (jnp.where(lane == 0, dist, 0.0)
            + jnp.where(lane == 1, rx, 0.0)
            + jnp.where(lane == 2, ry, 0.0)
            + jnp.where(lane == 3, rz, 0.0)
            + jnp.where(lane == 4, axy, 0.0)
            + jnp.where(lane == 5, axz, 0.0)
            + jnp.where(lane == 6, ayz, 0.0)
            + jnp.where(lane == 7, nx, 0.0)
            + jnp.where(lane == 8, ny, 0.0)
            + jnp.where(lane == 9, nz, 0.0))

    h = jnp.dot(feat, w1_ref[...], preferred_element_type=jnp.float32)
    h = jnp.maximum(_ln(h + b1_ref[...], g1_ref[...], be1_ref[...]), 0.0)
    h = jnp.dot(h, w2_ref[...], preferred_element_type=jnp.float32)
    h = jnp.maximum(_ln(h + b2_ref[...], g2_ref[...], be2_ref[...]), 0.0)
    h = jnp.dot(h, w3_ref[...], preferred_element_type=jnp.float32)
    h = _ln(h + b3_ref[...], g3_ref[...], be3_ref[...])

    P = jnp.transpose(E) * (1.0 / K)
    agg = jnp.dot(P, h, preferred_element_type=jnp.float32,
                  precision=jax.lax.Precision.HIGHEST)

    a = jnp.dot(agg, wa1_ref[...], preferred_element_type=jnp.float32)
    a = jnp.maximum(_ln(a + ba1_ref[...], ga1_ref[...], bea1_ref[...]), 0.0)
    out_ref[...] = (jnp.dot(a, wa2_ref[...], preferred_element_type=jnp.float32)
                    + ba2_ref[...])


def _mlp(nbr, table, K, PT, weights):
    (W1p, b1, g1, be1, W2, b2, g2, be2, W3, b3, g3, be3,
     Wa1, ba1, ga1, bea1, Wa2, ba2) = weights
    BN = table.shape[0]
    RT = PT * K
    grid = (BN // PT,)

    def full(a):
        return pl.BlockSpec(a.shape, lambda i: (0,) * a.ndim)

    return pl.pallas_call(
        functools.partial(_mlp_body, K),
        grid=grid,
        in_specs=[
            pl.BlockSpec((RT, 16), lambda i: (i, 0)),
            pl.BlockSpec((PT, 16), lambda i: (i, 0)),
            full(W1p), full(b1), full(g1), full(be1),
            full(W2), full(b2), full(g2), full(be2),
            full(W3), full(b3), full(g3), full(be3),
            full(Wa1), full(ba1), full(ga1), full(bea1),
            full(Wa2), full(ba2),
        ],
        out_specs=pl.BlockSpec((PT, 128), lambda i: (i, 0)),
        out_shape=jax.ShapeDtypeStruct((BN, 128), jnp.float32),
    )(nbr, table, W1p, b1, g1, be1, W2, b2, g2, be2, W3, b3, g3, be3,
      Wa1, ba1, ga1, bea1, Wa2, ba2)


def kernel(coordinates, W1, b1, g1, be1, W2, b2, g2, be2, W3, b3, g3, be3,
           Wa1, ba1, ga1, bea1, Wa2, ba2):
    K = 16
    B, N, _ = coordinates.shape
    D = W3.shape[1]
    Q = 256 if N % 256 == 0 else N

    cpad = jnp.pad(coordinates, ((0, 0), (0, 0), (0, 5)))
    ct = jnp.transpose(cpad, (0, 2, 1))
    idx = _topk(cpad, ct, K, Q)

    table = jnp.pad(coordinates.reshape(B * N, 3), ((0, 0), (0, 13)))
    NW = 32
    gidx = idx.reshape(NW, (B * N * K) // (NW * 128), 128)
    nbr = _sc_gather(table, gidx)

    W1p = jnp.pad(W1, ((0, 6), (0, 0)))
    weights = (W1p, b1.reshape(1, -1), g1.reshape(1, -1), be1.reshape(1, -1),
               W2, b2.reshape(1, -1), g2.reshape(1, -1), be2.reshape(1, -1),
               W3, b3.reshape(1, -1), g3.reshape(1, -1), be3.reshape(1, -1),
               Wa1, ba1.reshape(1, -1), ga1.reshape(1, -1), bea1.reshape(1, -1),
               Wa2, ba2.reshape(1, -1))
    out = _mlp(nbr, table, K, 64, weights)
    return out.reshape(B, N, D)


# speedup vs baseline: 7.0861x; 1.1352x over previous
import functools

import jax
import jax.numpy as jnp
from jax import lax
from jax.experimental import pallas as pl
from jax.experimental.pallas import tpu as pltpu
from jax.experimental.pallas import tpu_sc as plsc


def _topk_body(K, N, cq_ref, ck_ref, idx_ref):
    b = pl.program_id(0)
    cq = cq_ref[0]
    ck = ck_ref[0]
    sqq = jnp.sum(cq * cq, axis=1, keepdims=True)
    sqk = jnp.sum(ck * ck, axis=0, keepdims=True)
    d2 = sqq + sqk - 2.0 * jnp.dot(cq, ck, preferred_element_type=jnp.float32)
    d2 = jnp.maximum(d2, 0.0)
    Q = d2.shape[0]
    col = lax.broadcasted_iota(jnp.int32, (Q, N), 1)
    big = jnp.float32(3.4e38)
    js = []
    for _ in range(K):
        m = jnp.min(d2, axis=1, keepdims=True)
        cand = jnp.where(d2 == m, col, N)
        j = jnp.min(cand, axis=1, keepdims=True)
        js.append(j)
        d2 = jnp.where(col == j, big, d2)
    idx_ref[0] = jnp.concatenate(js, axis=1) + b * N


def _topk(coords_pad, coords_t, K, Q):
    B, N, _ = coords_pad.shape
    grid = (B, N // Q)
    return pl.pallas_call(
        functools.partial(_topk_body, K, N),
        grid=grid,
        in_specs=[
            pl.BlockSpec((1, Q, 8), lambda b, i: (b, i, 0)),
            pl.BlockSpec((1, 8, N), lambda b, i: (b, 0, 0)),
        ],
        out_specs=pl.BlockSpec((1, Q, K), lambda b, i: (b, i, 0)),
        out_shape=jax.ShapeDtypeStruct((B, N, K), jnp.int32),
    )(coords_pad, coords_t)


def _sc_gather(table, gidx):
    info = plsc.get_sparse_core_info()
    NC, NS = info.num_cores, info.num_subcores
    NW = NC * NS
    _, C, _ = gidx.shape
    rows_per_w = C * 128
    R = NW * rows_per_w
    mesh = plsc.VectorSubcoreMesh(core_axis_name="c", subcore_axis_name="s")

    @functools.partial(
        pl.kernel,
        mesh=mesh,
        compiler_params=pltpu.CompilerParams(use_tc_tiling_on_sc=False),
        out_type=jax.ShapeDtypeStruct((R, 16), jnp.float32),
        scratch_types=[
            pltpu.VMEM((C, 128), jnp.int32),
            pltpu.VMEM((rows_per_w, 16), jnp.float32),
            pltpu.SemaphoreType.DMA,
        ],
    )
    def k2(table_hbm, idx_hbm, out_hbm, idx_v, rows_v, sem):
        wid = lax.axis_index("s") * NC + lax.axis_index("c")
        pltpu.sync_copy(idx_hbm.at[wid], idx_v)

        def chunk(c, carry):
            pltpu.async_copy(
                table_hbm.at[idx_v.at[c]],
                rows_v.at[pl.ds(c * 128, 128)],
                sem,
            ).wait()
            return carry

        lax.fori_loop(0, C, chunk, 0)
        pltpu.sync_copy(rows_v, out_hbm.at[pl.ds(wid * rows_per_w, rows_per_w)])

    return k2(table, gidx)


def _ln(h, g, b, eps=1e-5):
    mu = jnp.mean(h, axis=-1, keepdims=True)
    var = jnp.mean((h - mu) ** 2, axis=-1, keepdims=True)
    return (h - mu) / jnp.sqrt(var + eps) * g + b


def _safe_atan2(y, x):
    both_zero = (jnp.abs(x) < 1e-9) & (jnp.abs(y) < 1e-9)
    x_safe = jnp.where(both_zero, 1.0, x)
    y_safe = jnp.where(both_zero, 0.0, y)
    return jnp.arctan2(y_safe, x_safe)


def _mlp_body(K, nbr_ref, ctr_ref,
              w1_ref, b1_ref, g1_ref, be1_ref,
              w2_ref, b2_ref, g2_ref, be2_ref,
              w3_ref, b3_ref, g3_ref, be3_ref,
              wa1_ref, ba1_ref, ga1_ref, bea1_ref,
              wa2_ref, ba2_ref, out_ref):
    nb = nbr_ref[...]
    ctr = ctr_ref[...]
    RT = nb.shape[0]
    PT = ctr.shape[0]

    ctr_rows = jnp.broadcast_to(ctr[:, None, :], (PT, K, 16)).reshape(RT, 16)

    rel = nb - ctr_rows
    d2 = jnp.sum(rel * rel, axis=1, keepdims=True)
    dist = jnp.sqrt(d2 + 1e-12)
    reln = rel / (dist + 1e-6)

    lane = lax.broadcasted_iota(jnp.int32, (RT, 16), 1)

    def col(a, c):
        return jnp.sum(jnp.where(lane == c, a, 0.0), axis=1, keepdims=True)

    rx, ry, rz = col(rel, 0), col(rel, 1), col(rel, 2)
    nx, ny, nz = col(reln, 0), col(reln, 1), col(reln, 2)
    axy = _safe_atan2(ny, nx)
    axz = _safe_atan2(nz, nx)
    ayz = _safe_atan2(nz, ny)

    feat = (jnp.where(lane == 0, dist, 0.0)
            + jnp.where(lane == 1, rx, 0.0)
            + jnp.where(lane == 2, ry, 0.0)
            + jnp.where(lane == 3, rz, 0.0)
            + jnp.where(lane == 4, axy, 0.0)
            + jnp.where(lane == 5, axz, 0.0)
            + jnp.where(lane == 6, ayz, 0.0)
            + jnp.where(lane == 7, nx, 0.0)
            + jnp.where(lane == 8, ny, 0.0)
            + jnp.where(lane == 9, nz, 0.0))

    h = jnp.dot(feat, w1_ref[...], preferred_element_type=jnp.float32)
    h = jnp.maximum(_ln(h + b1_ref[...], g1_ref[...], be1_ref[...]), 0.0)
    h = jnp.dot(h, w2_ref[...], preferred_element_type=jnp.float32)
    h = jnp.maximum(_ln(h + b2_ref[...], g2_ref[...], be2_ref[...]), 0.0)
    h = jnp.dot(h, w3_ref[...], preferred_element_type=jnp.float32)
    h = _ln(h + b3_ref[...], g3_ref[...], be3_ref[...])

    agg = jnp.mean(h.reshape(PT, K, 128), axis=1)

    a = jnp.dot(agg, wa1_ref[...], preferred_element_type=jnp.float32)
    a = jnp.maximum(_ln(a + ba1_ref[...], ga1_ref[...], bea1_ref[...]), 0.0)
    out_ref[...] = (jnp.dot(a, wa2_ref[...], preferred_element_type=jnp.float32)
                    + ba2_ref[...])


def _mlp(nbr, table, K, PT, weights):
    (W1p, b1, g1, be1, W2, b2, g2, be2, W3, b3, g3, be3,
     Wa1, ba1, ga1, bea1, Wa2, ba2) = weights
    BN = table.shape[0]
    RT = PT * K
    grid = (BN // PT,)

    def full(a):
        return pl.BlockSpec(a.shape, lambda i: (0,) * a.ndim)

    return pl.pallas_call(
        functools.partial(_mlp_body, K),
        grid=grid,
        in_specs=[
            pl.BlockSpec((RT, 16), lambda i: (i, 0)),
            pl.BlockSpec((PT, 16), lambda i: (i, 0)),
            full(W1p), full(b1), full(g1), full(be1),
            full(W2), full(b2), full(g2), full(be2),
            full(W3), full(b3), full(g3), full(be3),
            full(Wa1), full(ba1), full(ga1), full(bea1),
            full(Wa2), full(ba2),
        ],
        out_specs=pl.BlockSpec((PT, 128), lambda i: (i, 0)),
        out_shape=jax.ShapeDtypeStruct((BN, 128), jnp.float32),
    )(nbr, table, W1p, b1, g1, be1, W2, b2, g2, be2, W3, b3, g3, be3,
      Wa1, ba1, ga1, bea1, Wa2, ba2)


def kernel(coordinates, W1, b1, g1, be1, W2, b2, g2, be2, W3, b3, g3, be3,
           Wa1, ba1, ga1, bea1, Wa2, ba2):
    K = 16
    B, N, _ = coordinates.shape
    D = W3.shape[1]
    Q = 256 if N % 256 == 0 else N

    cpad = jnp.pad(coordinates, ((0, 0), (0, 0), (0, 5)))
    ct = jnp.transpose(cpad, (0, 2, 1))
    idx = _topk(cpad, ct, K, Q)

    table = jnp.pad(coordinates.reshape(B * N, 3), ((0, 0), (0, 13)))
    NW = 32
    gidx = idx.reshape(NW, (B * N * K) // (NW * 128), 128)
    nbr = _sc_gather(table, gidx)

    W1p = jnp.pad(W1, ((0, 6), (0, 0)))
    weights = (W1p, b1.reshape(1, -1), g1.reshape(1, -1), be1.reshape(1, -1),
               W2, b2.reshape(1, -1), g2.reshape(1, -1), be2.reshape(1, -1),
               W3, b3.reshape(1, -1), g3.reshape(1, -1), be3.reshape(1, -1),
               Wa1, ba1.reshape(1, -1), ga1.reshape(1, -1), bea1.reshape(1, -1),
               Wa2, ba2.reshape(1, -1))
    out = _mlp(nbr, table, K, 64, weights)
    return out.reshape(B, N, D)


# speedup vs baseline: 8.3395x; 1.1769x over previous
import functools

import jax
import jax.numpy as jnp
from jax import lax
from jax.experimental import pallas as pl
from jax.experimental.pallas import tpu as pltpu
from jax.experimental.pallas import tpu_sc as plsc


def _topk_body(K, N, cq_ref, ck_ref, idx_ref):
    b = pl.program_id(0)
    cq = cq_ref[0]
    ck = ck_ref[0]
    sqq = jnp.sum(cq * cq, axis=1, keepdims=True)
    sqk = jnp.sum(ck * ck, axis=0, keepdims=True)
    d2 = sqq + sqk - 2.0 * jnp.dot(cq, ck, preferred_element_type=jnp.float32)
    d2 = jnp.maximum(d2, 0.0)
    Q = d2.shape[0]
    col = lax.broadcasted_iota(jnp.int32, (Q, N), 1)
    big = jnp.float32(3.4e38)
    js = []
    for _ in range(K):
        m = jnp.min(d2, axis=1, keepdims=True)
        cand = jnp.where(d2 == m, col, N)
        j = jnp.min(cand, axis=1, keepdims=True)
        js.append(j)
        d2 = jnp.where(col == j, big, d2)
    idx_ref[0] = jnp.concatenate(js, axis=1) + b * N


def _topk(coords_pad, coords_t, K, Q):
    B, N, _ = coords_pad.shape
    grid = (B, N // Q)
    return pl.pallas_call(
        functools.partial(_topk_body, K, N),
        grid=grid,
        in_specs=[
            pl.BlockSpec((1, Q, 8), lambda b, i: (b, i, 0)),
            pl.BlockSpec((1, 8, N), lambda b, i: (b, 0, 0)),
        ],
        out_specs=pl.BlockSpec((1, Q, K), lambda b, i: (b, i, 0)),
        out_shape=jax.ShapeDtypeStruct((B, N, K), jnp.int32),
    )(coords_pad, coords_t)


def _sc_gather(table, gidx):
    info = plsc.get_sparse_core_info()
    NC, NS = info.num_cores, info.num_subcores
    NW = NC * NS
    _, C, _ = gidx.shape
    rows_per_w = C * 128
    R = NW * rows_per_w
    mesh = plsc.VectorSubcoreMesh(core_axis_name="c", subcore_axis_name="s")

    @functools.partial(
        pl.kernel,
        mesh=mesh,
        compiler_params=pltpu.CompilerParams(use_tc_tiling_on_sc=False),
        out_type=jax.ShapeDtypeStruct((R, 16), jnp.float32),
        scratch_types=[
            pltpu.VMEM((C, 128), jnp.int32),
            pltpu.VMEM((rows_per_w, 16), jnp.float32),
            pltpu.SemaphoreType.DMA,
        ],
    )
    def k2(table_hbm, idx_hbm, out_hbm, idx_v, rows_v, sem):
        wid = lax.axis_index("s") * NC + lax.axis_index("c")
        pltpu.sync_copy(idx_hbm.at[wid], idx_v)

        def chunk(c, carry):
            pltpu.async_copy(
                table_hbm.at[idx_v.at[c]],
                rows_v.at[pl.ds(c * 128, 128)],
                sem,
            ).wait()
            return carry

        lax.fori_loop(0, C, chunk, 0)
        pltpu.sync_copy(rows_v, out_hbm.at[pl.ds(wid * rows_per_w, rows_per_w)])

    return k2(table, gidx)


def _ln(h, g, b, eps=1e-5):
    mu = jnp.mean(h, axis=-1, keepdims=True)
    var = jnp.mean((h - mu) ** 2, axis=-1, keepdims=True)
    return (h - mu) / jnp.sqrt(var + eps) * g + b


def _safe_atan2(y, x):
    both_zero = (jnp.abs(x) < 1e-9) & (jnp.abs(y) < 1e-9)
    x_safe = jnp.where(both_zero, 1.0, x)
    y_safe = jnp.where(both_zero, 0.0, y)
    return jnp.arctan2(y_safe, x_safe)


def _mlp_body(K, nbr_ref, ctr_ref,
              w1_ref, b1_ref, g1_ref, be1_ref,
              w2_ref, b2_ref, g2_ref, be2_ref,
              w3_ref, b3_ref, g3_ref, be3_ref,
              wa1_ref, ba1_ref, ga1_ref, bea1_ref,
              wa2_ref, ba2_ref, out_ref):
    nb = nbr_ref[...]
    ctr = ctr_ref[...]
    RT = nb.shape[0]
    PT = ctr.shape[0]

    ctr_rows = jnp.broadcast_to(ctr[:, None, :], (PT, K, 16)).reshape(RT, 16)

    rel = nb - ctr_rows
    ones = jnp.ones((16, 8), jnp.float32)
    d2 = jnp.dot(rel * rel, ones, preferred_element_type=jnp.float32,
                 precision=jax.lax.Precision.HIGHEST)[:, :1]
    dist = jnp.sqrt(d2 + 1e-12)
    reln = rel / (dist + 1e-6)

    rel3 = rel[:, :3]
    reln3 = reln[:, :3]
    nx, ny, nz = reln[:, 0:1], reln[:, 1:2], reln[:, 2:3]
    ypack = jnp.concatenate([ny, nz, nz], axis=1)
    xpack = jnp.concatenate([nx, nx, ny], axis=1)
    ang = _safe_atan2(ypack, xpack)

    zeros6 = jnp.zeros((RT, 6), jnp.float32)
    feat = jnp.concatenate([dist, rel3, ang, reln3, zeros6], axis=1)

    h = jnp.dot(feat, w1_ref[...], preferred_element_type=jnp.float32)
    h = jnp.maximum(_ln(h + b1_ref[...], g1_ref[...], be1_ref[...]), 0.0)
    h = jnp.dot(h, w2_ref[...], preferred_element_type=jnp.float32)
    h = jnp.maximum(_ln(h + b2_ref[...], g2_ref[...], be2_ref[...]), 0.0)
    h = jnp.dot(h, w3_ref[...], preferred_element_type=jnp.float32)
    h = _ln(h + b3_ref[...], g3_ref[...], be3_ref[...])

    agg = jnp.mean(h.reshape(PT, K, 128), axis=1)

    a = jnp.dot(agg, wa1_ref[...], preferred_element_type=jnp.float32)
    a = jnp.maximum(_ln(a + ba1_ref[...], ga1_ref[...], bea1_ref[...]), 0.0)
    out_ref[...] = (jnp.dot(a, wa2_ref[...], preferred_element_type=jnp.float32)
                    + ba2_ref[...])


def _mlp(nbr, table, K, PT, weights):
    (W1p, b1, g1, be1, W2, b2, g2, be2, W3, b3, g3, be3,
     Wa1, ba1, ga1, bea1, Wa2, ba2) = weights
    BN = table.shape[0]
    RT = PT * K
    grid = (BN // PT,)

    def full(a):
        return pl.BlockSpec(a.shape, lambda i: (0,) * a.ndim)

    return pl.pallas_call(
        functools.partial(_mlp_body, K),
        grid=grid,
        in_specs=[
            pl.BlockSpec((RT, 16), lambda i: (i, 0)),
            pl.BlockSpec((PT, 16), lambda i: (i, 0)),
            full(W1p), full(b1), full(g1), full(be1),
            full(W2), full(b2), full(g2), full(be2),
            full(W3), full(b3), full(g3), full(be3),
            full(Wa1), full(ba1), full(ga1), full(bea1),
            full(Wa2), full(ba2),
        ],
        out_specs=pl.BlockSpec((PT, 128), lambda i: (i, 0)),
        out_shape=jax.ShapeDtypeStruct((BN, 128), jnp.float32),
    )(nbr, table, W1p, b1, g1, be1, W2, b2, g2, be2, W3, b3, g3, be3,
      Wa1, ba1, ga1, bea1, Wa2, ba2)


def kernel(coordinates, W1, b1, g1, be1, W2, b2, g2, be2, W3, b3, g3, be3,
           Wa1, ba1, ga1, bea1, Wa2, ba2):
    K = 16
    B, N, _ = coordinates.shape
    D = W3.shape[1]
    Q = 256 if N % 256 == 0 else N

    cpad = jnp.pad(coordinates, ((0, 0), (0, 0), (0, 5)))
    ct = jnp.transpose(cpad, (0, 2, 1))
    idx = _topk(cpad, ct, K, Q)

    table = jnp.pad(coordinates.reshape(B * N, 3), ((0, 0), (0, 13)))
    NW = 32
    gidx = idx.reshape(NW, (B * N * K) // (NW * 128), 128)
    nbr = _sc_gather(table, gidx)

    W1p = jnp.pad(W1, ((0, 6), (0, 0)))
    weights = (W1p, b1.reshape(1, -1), g1.reshape(1, -1), be1.reshape(1, -1),
               W2, b2.reshape(1, -1), g2.reshape(1, -1), be2.reshape(1, -1),
               W3, b3.reshape(1, -1), g3.reshape(1, -1), be3.reshape(1, -1),
               Wa1, ba1.reshape(1, -1), ga1.reshape(1, -1), bea1.reshape(1, -1),
               Wa2, ba2.reshape(1, -1))
    out = _mlp(nbr, table, K, 64, weights)
    return out.reshape(B, N, D)


# speedup vs baseline: 9.3768x; 1.1244x over previous
import functools

import jax
import jax.numpy as jnp
from jax import lax
from jax.experimental import pallas as pl
from jax.experimental.pallas import tpu as pltpu
from jax.experimental.pallas import tpu_sc as plsc


def _topk_body(K, N, cq_ref, ck_ref, idx_ref):
    b = pl.program_id(0)
    cq = cq_ref[0]
    ck = ck_ref[0]
    sqq = jnp.sum(cq * cq, axis=1, keepdims=True)
    sqk = jnp.sum(ck * ck, axis=0, keepdims=True)
    d2 = sqq + sqk - 2.0 * jnp.dot(cq, ck, preferred_element_type=jnp.float32)
    d2 = jnp.maximum(d2, 0.0)
    Q = d2.shape[0]
    col = lax.broadcasted_iota(jnp.int32, (Q, N), 1).astype(jnp.float32)
    big = jnp.float32(3.4e38)
    fn = jnp.float32(N)
    js = []
    for _ in range(K):
        m = jnp.min(d2, axis=1, keepdims=True)
        cand = jnp.where(d2 == m, col, fn)
        j = jnp.min(cand, axis=1, keepdims=True)
        js.append(j)
        d2 = jnp.where(col == j, big, d2)
    idx = jnp.concatenate(js, axis=1).astype(jnp.int32)
    idx_ref[0] = idx + b * N


def _topk(coords_pad, coords_t, K, Q):
    B, N, _ = coords_pad.shape
    grid = (B, N // Q)
    return pl.pallas_call(
        functools.partial(_topk_body, K, N),
        grid=grid,
        in_specs=[
            pl.BlockSpec((1, Q, 8), lambda b, i: (b, i, 0)),
            pl.BlockSpec((1, 8, N), lambda b, i: (b, 0, 0)),
        ],
        out_specs=pl.BlockSpec((1, Q, K), lambda b, i: (b, i, 0)),
        out_shape=jax.ShapeDtypeStruct((B, N, K), jnp.int32),
    )(coords_pad, coords_t)


def _sc_gather(table, gidx):
    info = plsc.get_sparse_core_info()
    NC, NS = info.num_cores, info.num_subcores
    NW = NC * NS
    _, C, _ = gidx.shape
    rows_per_w = C * 128
    R = NW * rows_per_w
    mesh = plsc.VectorSubcoreMesh(core_axis_name="c", subcore_axis_name="s")

    @functools.partial(
        pl.kernel,
        mesh=mesh,
        compiler_params=pltpu.CompilerParams(use_tc_tiling_on_sc=False),
        out_type=jax.ShapeDtypeStruct((R, 16), jnp.float32),
        scratch_types=[
            pltpu.VMEM((C, 128), jnp.int32),
            pltpu.VMEM((rows_per_w, 16), jnp.float32),
            pltpu.SemaphoreType.DMA,
        ],
    )
    def k2(table_hbm, idx_hbm, out_hbm, idx_v, rows_v, sem):
        wid = lax.axis_index("s") * NC + lax.axis_index("c")
        pltpu.sync_copy(idx_hbm.at[wid], idx_v)

        def chunk(c, carry):
            pltpu.async_copy(
                table_hbm.at[idx_v.at[c]],
                rows_v.at[pl.ds(c * 128, 128)],
                sem,
            ).wait()
            return carry

        lax.fori_loop(0, C, chunk, 0)
        pltpu.sync_copy(rows_v, out_hbm.at[pl.ds(wid * rows_per_w, rows_per_w)])

    return k2(table, gidx)


def _ln(h, g, b, eps=1e-5):
    mu = jnp.mean(h, axis=-1, keepdims=True)
    var = jnp.mean((h - mu) ** 2, axis=-1, keepdims=True)
    return (h - mu) / jnp.sqrt(var + eps) * g + b


def _safe_atan2(y, x):
    both_zero = (jnp.abs(x) < 1e-9) & (jnp.abs(y) < 1e-9)
    x_safe = jnp.where(both_zero, 1.0, x)
    y_safe = jnp.where(both_zero, 0.0, y)
    return jnp.arctan2(y_safe, x_safe)


def _mlp_body(K, nbr_ref, ctr_ref,
              w1_ref, b1_ref, g1_ref, be1_ref,
              w2_ref, b2_ref, g2_ref, be2_ref,
              w3_ref, b3_ref, g3_ref, be3_ref,
              wa1_ref, ba1_ref, ga1_ref, bea1_ref,
              wa2_ref, ba2_ref, out_ref):
    nb = nbr_ref[...]
    ctr = ctr_ref[...]
    RT = nb.shape[0]
    PT = ctr.shape[0]

    ctr_rows = jnp.broadcast_to(ctr[:, None, :], (PT, K, 16)).reshape(RT, 16)

    rel = nb - ctr_rows
    ones = jnp.ones((16, 8), jnp.float32)
    d2 = jnp.dot(rel * rel, ones, preferred_element_type=jnp.float32,
                 precision=jax.lax.Precision.HIGHEST)[:, :1]
    dist = jnp.sqrt(d2 + 1e-12)
    reln = rel / (dist + 1e-6)

    rel3 = rel[:, :3]
    reln3 = reln[:, :3]
    nx, ny, nz = reln[:, 0:1], reln[:, 1:2], reln[:, 2:3]
    ypack = jnp.concatenate([ny, nz, nz], axis=1)
    xpack = jnp.concatenate([nx, nx, ny], axis=1)
    ang = _safe_atan2(ypack, xpack)

    zeros6 = jnp.zeros((RT, 6), jnp.float32)
    feat = jnp.concatenate([dist, rel3, ang, reln3, zeros6], axis=1)

    h = jnp.dot(feat, w1_ref[...], preferred_element_type=jnp.float32)
    h = jnp.maximum(_ln(h + b1_ref[...], g1_ref[...], be1_ref[...]), 0.0)
    h = jnp.dot(h, w2_ref[...], preferred_element_type=jnp.float32)
    h = jnp.maximum(_ln(h + b2_ref[...], g2_ref[...], be2_ref[...]), 0.0)
    h = jnp.dot(h, w3_ref[...], preferred_element_type=jnp.float32)
    h = _ln(h + b3_ref[...], g3_ref[...], be3_ref[...])

    agg = jnp.mean(h.reshape(PT, K, 128), axis=1)

    a = jnp.dot(agg, wa1_ref[...], preferred_element_type=jnp.float32)
    a = jnp.maximum(_ln(a + ba1_ref[...], ga1_ref[...], bea1_ref[...]), 0.0)
    out_ref[...] = (jnp.dot(a, wa2_ref[...], preferred_element_type=jnp.float32)
                    + ba2_ref[...])


def _mlp(nbr, table, K, PT, weights):
    (W1p, b1, g1, be1, W2, b2, g2, be2, W3, b3, g3, be3,
     Wa1, ba1, ga1, bea1, Wa2, ba2) = weights
    BN = table.shape[0]
    RT = PT * K
    grid = (BN // PT,)

    def full(a):
        return pl.BlockSpec(a.shape, lambda i: (0,) * a.ndim)

    return pl.pallas_call(
        functools.partial(_mlp_body, K),
        grid=grid,
        in_specs=[
            pl.BlockSpec((RT, 16), lambda i: (i, 0)),
            pl.BlockSpec((PT, 16), lambda i: (i, 0)),
            full(W1p), full(b1), full(g1), full(be1),
            full(W2), full(b2), full(g2), full(be2),
            full(W3), full(b3), full(g3), full(be3),
            full(Wa1), full(ba1), full(ga1), full(bea1),
            full(Wa2), full(ba2),
        ],
        out_specs=pl.BlockSpec((PT, 128), lambda i: (i, 0)),
        out_shape=jax.ShapeDtypeStruct((BN, 128), jnp.float32),
    )(nbr, table, W1p, b1, g1, be1, W2, b2, g2, be2, W3, b3, g3, be3,
      Wa1, ba1, ga1, bea1, Wa2, ba2)


def kernel(coordinates, W1, b1, g1, be1, W2, b2, g2, be2, W3, b3, g3, be3,
           Wa1, ba1, ga1, bea1, Wa2, ba2):
    K = 16
    B, N, _ = coordinates.shape
    D = W3.shape[1]
    Q = 256 if N % 256 == 0 else N

    cpad = jnp.pad(coordinates, ((0, 0), (0, 0), (0, 5)))
    ct = jnp.transpose(cpad, (0, 2, 1))
    idx = _topk(cpad, ct, K, Q)

    table = jnp.pad(coordinates.reshape(B * N, 3), ((0, 0), (0, 13)))
    NW = 32
    gidx = idx.reshape(NW, (B * N * K) // (NW * 128), 128)
    nbr = _sc_gather(table, gidx)

    W1p = jnp.pad(W1, ((0, 6), (0, 0)))
    weights = (W1p, b1.reshape(1, -1), g1.reshape(1, -1), be1.reshape(1, -1),
               W2, b2.reshape(1, -1), g2.reshape(1, -1), be2.reshape(1, -1),
               W3, b3.reshape(1, -1), g3.reshape(1, -1), be3.reshape(1, -1),
               Wa1, ba1.reshape(1, -1), ga1.reshape(1, -1), bea1.reshape(1, -1),
               Wa2, ba2.reshape(1, -1))
    out = _mlp(nbr, table, K, 64, weights)
    return out.reshape(B, N, D)


# speedup vs baseline: 9.4162x; 1.0042x over previous
import functools

import jax
import jax.numpy as jnp
from jax import lax
from jax.experimental import pallas as pl
from jax.experimental.pallas import tpu as pltpu
from jax.experimental.pallas import tpu_sc as plsc


def _topk_body(K, N, cq_ref, ck_ref, idx_ref):
    b = pl.program_id(0)
    cq = cq_ref[0]
    ck = ck_ref[0]
    sqq = jnp.sum(cq * cq, axis=1, keepdims=True)
    sqk = jnp.sum(ck * ck, axis=0, keepdims=True)
    d2 = sqq + sqk - 2.0 * jnp.dot(cq, ck, preferred_element_type=jnp.float32)
    d2 = jnp.maximum(d2, 0.0)
    Q = d2.shape[0]
    col = lax.broadcasted_iota(jnp.int32, (Q, N), 1).astype(jnp.float32)
    big = jnp.float32(3.4e38)
    fn = jnp.float32(N)
    js = []
    for _ in range(K):
        m = jnp.min(d2, axis=1, keepdims=True)
        cand = jnp.where(d2 == m, col, fn)
        j = jnp.min(cand, axis=1, keepdims=True)
        js.append(j)
        d2 = jnp.where(col == j, big, d2)
    idx = jnp.concatenate(js, axis=1).astype(jnp.int32)
    idx_ref[0] = idx + b * N


def _topk(coords_pad, coords_t, K, Q):
    B, N, _ = coords_pad.shape
    grid = (B, N // Q)
    return pl.pallas_call(
        functools.partial(_topk_body, K, N),
        grid=grid,
        in_specs=[
            pl.BlockSpec((1, Q, 8), lambda b, i: (b, i, 0)),
            pl.BlockSpec((1, 8, N), lambda b, i: (b, 0, 0)),
        ],
        out_specs=pl.BlockSpec((1, Q, K), lambda b, i: (b, i, 0)),
        out_shape=jax.ShapeDtypeStruct((B, N, K), jnp.int32),
    )(coords_pad, coords_t)


def _sc_gather(table, gidx):
    info = plsc.get_sparse_core_info()
    NC, NS = info.num_cores, info.num_subcores
    NW = NC * NS
    _, C, _ = gidx.shape
    rows_per_w = C * 128
    R = NW * rows_per_w
    mesh = plsc.VectorSubcoreMesh(core_axis_name="c", subcore_axis_name="s")

    @functools.partial(
        pl.kernel,
        mesh=mesh,
        compiler_params=pltpu.CompilerParams(use_tc_tiling_on_sc=False),
        out_type=jax.ShapeDtypeStruct((R, 16), jnp.float32),
        scratch_types=[
            pltpu.VMEM((C, 128), jnp.int32),
            pltpu.VMEM((rows_per_w, 16), jnp.float32),
            pltpu.SemaphoreType.DMA,
        ],
    )
    def k2(table_hbm, idx_hbm, out_hbm, idx_v, rows_v, sem):
        wid = lax.axis_index("s") * NC + lax.axis_index("c")
        pltpu.sync_copy(idx_hbm.at[wid], idx_v)

        def chunk(c, carry):
            pltpu.async_copy(
                table_hbm.at[idx_v.at[c]],
                rows_v.at[pl.ds(c * 128, 128)],
                sem,
            ).wait()
            return carry

        lax.fori_loop(0, C, chunk, 0)
        pltpu.sync_copy(rows_v, out_hbm.at[pl.ds(wid * rows_per_w, rows_per_w)])

    return k2(table, gidx)


def _ln(h, g, b, eps=1e-5):
    mu = jnp.mean(h, axis=-1, keepdims=True)
    var = jnp.mean((h - mu) ** 2, axis=-1, keepdims=True)
    return (h - mu) / jnp.sqrt(var + eps) * g + b


def _safe_atan2(y, x):
    both_zero = (jnp.abs(x) < 1e-9) & (jnp.abs(y) < 1e-9)
    x_safe = jnp.where(both_zero, 1.0, x)
    y_safe = jnp.where(both_zero, 0.0, y)
    return jnp.arctan2(y_safe, x_safe)


def _mlp_body(K, nbr_ref, ctr_ref,
              w1_ref, b1_ref, g1_ref, be1_ref,
              w2_ref, b2_ref, g2_ref, be2_ref,
              w3_ref, b3_ref, g3_ref, be3_ref,
              wa1_ref, ba1_ref, ga1_ref, bea1_ref,
              wa2_ref, ba2_ref, out_ref):
    nb = nbr_ref[...]
    ctr = ctr_ref[...]
    RT = nb.shape[0]
    PT = ctr.shape[0]

    ctr_rows = jnp.broadcast_to(ctr[:, None, :], (PT, K, 16)).reshape(RT, 16)

    rel = nb - ctr_rows
    ones = jnp.ones((16, 8), jnp.float32)
    d2 = jnp.dot(rel * rel, ones, preferred_element_type=jnp.float32,
                 precision=jax.lax.Precision.HIGHEST)[:, :1]
    dist = jnp.sqrt(d2 + 1e-12)
    reln = rel / (dist + 1e-6)

    rel3 = rel[:, :3]
    reln3 = reln[:, :3]
    nx, ny, nz = reln[:, 0:1], reln[:, 1:2], reln[:, 2:3]
    ypack = jnp.concatenate([ny, nz, nz], axis=1)
    xpack = jnp.concatenate([nx, nx, ny], axis=1)
    ang = _safe_atan2(ypack, xpack)

    zeros6 = jnp.zeros((RT, 6), jnp.float32)
    feat = jnp.concatenate([dist, rel3, ang, reln3, zeros6], axis=1)

    h = jnp.dot(feat, w1_ref[...], preferred_element_type=jnp.float32)
    h = jnp.maximum(_ln(h + b1_ref[...], g1_ref[...], be1_ref[...]), 0.0)
    h = jnp.dot(h, w2_ref[...], preferred_element_type=jnp.float32)
    h = jnp.maximum(_ln(h + b2_ref[...], g2_ref[...], be2_ref[...]), 0.0)
    h = jnp.dot(h, w3_ref[...], preferred_element_type=jnp.float32)
    h = _ln(h + b3_ref[...], g3_ref[...], be3_ref[...])

    agg = jnp.mean(h.reshape(PT, K, 128), axis=1)

    a = jnp.dot(agg, wa1_ref[...], preferred_element_type=jnp.float32)
    a = jnp.maximum(_ln(a + ba1_ref[...], ga1_ref[...], bea1_ref[...]), 0.0)
    out_ref[...] = (jnp.dot(a, wa2_ref[...], preferred_element_type=jnp.float32)
                    + ba2_ref[...])


def _mlp(nbr, table, K, PT, weights):
    (W1p, b1, g1, be1, W2, b2, g2, be2, W3, b3, g3, be3,
     Wa1, ba1, ga1, bea1, Wa2, ba2) = weights
    BN = table.shape[0]
    RT = PT * K
    grid = (BN // PT,)

    def full(a):
        return pl.BlockSpec(a.shape, lambda i: (0,) * a.ndim)

    return pl.pallas_call(
        functools.partial(_mlp_body, K),
        grid=grid,
        in_specs=[
            pl.BlockSpec((RT, 16), lambda i: (i, 0)),
            pl.BlockSpec((PT, 16), lambda i: (i, 0)),
            full(W1p), full(b1), full(g1), full(be1),
            full(W2), full(b2), full(g2), full(be2),
            full(W3), full(b3), full(g3), full(be3),
            full(Wa1), full(ba1), full(ga1), full(bea1),
            full(Wa2), full(ba2),
        ],
        out_specs=pl.BlockSpec((PT, 128), lambda i: (i, 0)),
        out_shape=jax.ShapeDtypeStruct((BN, 128), jnp.float32),
    )(nbr, table, W1p, b1, g1, be1, W2, b2, g2, be2, W3, b3, g3, be3,
      Wa1, ba1, ga1, bea1, Wa2, ba2)


def kernel(coordinates, W1, b1, g1, be1, W2, b2, g2, be2, W3, b3, g3, be3,
           Wa1, ba1, ga1, bea1, Wa2, ba2):
    K = 16
    B, N, _ = coordinates.shape
    D = W3.shape[1]
    Q = 256 if N % 256 == 0 else N

    cpad = jnp.pad(coordinates, ((0, 0), (0, 0), (0, 5)))
    ct = jnp.transpose(cpad, (0, 2, 1))
    table = jnp.pad(coordinates.reshape(B * N, 3), ((0, 0), (0, 13)))
    NW = 32

    W1p = jnp.pad(W1, ((0, 6), (0, 0)))
    weights = (W1p, b1.reshape(1, -1), g1.reshape(1, -1), be1.reshape(1, -1),
               W2, b2.reshape(1, -1), g2.reshape(1, -1), be2.reshape(1, -1),
               W3, b3.reshape(1, -1), g3.reshape(1, -1), be3.reshape(1, -1),
               Wa1, ba1.reshape(1, -1), ga1.reshape(1, -1), bea1.reshape(1, -1),
               Wa2, ba2.reshape(1, -1))

    idxs = [_topk(cpad[b:b + 1], ct[b:b + 1], K, Q) for b in range(B)]
    nbrs = [_sc_gather(table, (idxs[b] + b * N).reshape(
        NW, (N * K) // (NW * 128), 128)) for b in range(B)]
    outs = [_mlp(nbrs[b], table[b * N:(b + 1) * N], K, 64, weights)
            for b in range(B)]
    return jnp.stack(outs).reshape(B, N, D)
